# Initial kernel scaffold; baseline (speedup 1.0000x reference)
import jax
import jax.numpy as jnp
from jax.experimental import pallas as pl


def kernel(ent_emb, rel_emb, norm_vec, pos_triplets, neg_triplets):
    raise NotImplementedError("write your pallas kernel here")


# speedup vs baseline: 1.2713x; 1.2713x over previous
import functools

import jax
import jax.numpy as jnp
from jax.experimental import pallas as pl
from jax.experimental.pallas import tpu as pltpu


def _transh_kernel(
    ph_idx, pt_idx, nh_idx, nt_idx, pr_idx, nr_idx,
    ent_hbm,
    relnorm_ref,
    out_ref,
    ent_vmem,
    pht, ptt, nht, ntt,
    prt, nrt,
    copy_sem,
    *, margin, alpha, batch, dim, rows_per_core, unroll):
  core = pl.program_id(0)
  off = core * rows_per_core
  n_chunks = rows_per_core // unroll

  cp = pltpu.make_async_copy(ent_hbm, ent_vmem, copy_sem)
  cp.start()

  def rel_body(c, carry):
    base = c * unroll
    for u in range(unroll):
      mi = base + u
      gi = off + mi
      prt[mi, 0] = relnorm_ref[pr_idx[gi], 0]
      nrt[mi, 0] = relnorm_ref[nr_idx[gi], 0]
    return carry
  jax.lax.fori_loop(0, n_chunks, rel_body, 0)

  cp.wait()

  def ent_body(c, carry):
    base = c * unroll
    for u in range(unroll):
      mi = base + u
      gi = off + mi
      pht[mi, 0] = ent_vmem[ph_idx[gi], 0]
      ptt[mi, 0] = ent_vmem[pt_idx[gi], 0]
      nht[mi, 0] = ent_vmem[nh_idx[gi], 0]
      ntt[mi, 0] = ent_vmem[nt_idx[gi], 0]
    return carry
  jax.lax.fori_loop(0, n_chunks, ent_body, 0)

  cchunk = 256 if rows_per_core % 256 == 0 else unroll
  n_cchunks = rows_per_core // cchunk

  def side(h, r, t, w):
    hw = jnp.sum(h * w, axis=2, keepdims=True)
    tw = jnp.sum(t * w, axis=2, keepdims=True)
    scores = (h + r - t) - (hw - tw) * w
    dist = jnp.sum(jnp.abs(scores), axis=2, keepdims=True)
    reg = jnp.sum(jnp.abs(h), axis=2, keepdims=True) + \
          jnp.sum(jnp.abs(t), axis=2, keepdims=True)
    r2 = jnp.sum(r * r, axis=2, keepdims=True)
    return dist, reg, r2

  def compute_body(c, carry):
    hinge_s, reg_s, r2_s = carry
    base = c * cchunk
    sl = pl.ds(base, cchunk)
    ph, pt = pht[sl], ptt[sl]
    nh, nt = nht[sl], ntt[sl]
    prw, nrw = prt[sl], nrt[sl]
    pr, pw = prw[:, :, :dim], prw[:, :, dim:]
    nr, nw = nrw[:, :, :dim], nrw[:, :, dim:]

    pd, p_reg, p_r2 = side(ph, pr, pt, pw)
    nd, n_reg, n_r2 = side(nh, nr, nt, nw)

    rows = (off + base
            + jax.lax.broadcasted_iota(jnp.int32, (cchunk, 1, 1), 0))
    mask = (rows < batch).astype(jnp.float32)
    hinge = jnp.maximum(pd - nd + margin, 0.0)
    return (hinge_s + jnp.sum(hinge * mask),
            reg_s + jnp.sum((p_reg + n_reg) * mask),
            r2_s + jnp.sum((p_r2 + n_r2) * mask))

  zero = jnp.float32(0.0)
  hinge_s, reg_s, r2_s = jax.lax.fori_loop(
      0, n_cchunks, compute_body, (zero, zero, zero))

  inv_b = 1.0 / batch
  s = (hinge_s * inv_b
       + (alpha / 3.0) * (reg_s * inv_b + r2_s * (inv_b / dim)))
  out_ref[...] = jnp.reshape(s, (1, 1, 1))


def _transh_loss(ent_emb, rel_emb, norm_vec, pos_triplets, neg_triplets,
                 *, margin=4.0, alpha=0.01):
  B = int(pos_triplets.shape[0])
  E, D = int(ent_emb.shape[0]), int(ent_emb.shape[1])
  R = int(rel_emb.shape[0])
  ncores = 2
  unroll = 8

  rows_per_core = pl.cdiv(B, ncores * unroll) * unroll
  padded = ncores * rows_per_core

  ent3 = ent_emb.astype(jnp.float32).reshape(E, 1, D)
  relnorm = jnp.concatenate(
      [rel_emb.astype(jnp.float32), norm_vec.astype(jnp.float32)],
      axis=1).reshape(R, 1, 2 * D)

  def col(trip, j):
    c = trip[:, j].astype(jnp.int32)
    return jnp.pad(c, (0, padded - B))

  ph, pr, pt = col(pos_triplets, 0), col(pos_triplets, 1), col(pos_triplets, 2)
  nh, nr, nt = col(neg_triplets, 0), col(neg_triplets, 1), col(neg_triplets, 2)

  tiles_bytes = rows_per_core * (4 * D + 2 * 2 * D) * 4
  vmem_bytes = (E * D + R * 2 * D) * 4 + tiles_bytes + (8 << 20)
  grid_spec = pltpu.PrefetchScalarGridSpec(
      num_scalar_prefetch=6,
      grid=(ncores,),
      in_specs=[pl.BlockSpec(memory_space=pl.ANY),
                pl.BlockSpec((R, 1, 2 * D), lambda c, *_: (0, 0, 0))],
      out_specs=pl.BlockSpec((1, 1, 1), lambda c, *_: (c, 0, 0)),
      scratch_shapes=[
          pltpu.VMEM((E, 1, D), jnp.float32),
          pltpu.VMEM((rows_per_core, 1, D), jnp.float32),
          pltpu.VMEM((rows_per_core, 1, D), jnp.float32),
          pltpu.VMEM((rows_per_core, 1, D), jnp.float32),
          pltpu.VMEM((rows_per_core, 1, D), jnp.float32),
          pltpu.VMEM((rows_per_core, 1, 2 * D), jnp.float32),
          pltpu.VMEM((rows_per_core, 1, 2 * D), jnp.float32),
          pltpu.SemaphoreType.DMA,
      ])
  out = pl.pallas_call(
      functools.partial(_transh_kernel, margin=float(margin),
                        alpha=float(alpha), batch=B, dim=D,
                        rows_per_core=rows_per_core, unroll=unroll),
      out_shape=jax.ShapeDtypeStruct((ncores, 1, 1), jnp.float32),
      grid_spec=grid_spec,
      compiler_params=pltpu.CompilerParams(
          dimension_semantics=("parallel",),
          vmem_limit_bytes=int(min(58 * 2**20, vmem_bytes))),
      cost_estimate=pl.CostEstimate(
          flops=2 * padded * D * 30,
          transcendentals=0,
          bytes_accessed=(2 * E * D + R * 2 * D + 4 * padded * D
                          + 6 * padded) * 4),
      name="transh_loss",
  )(ph, pt, nh, nt, pr, nr, ent3, relnorm)

  return out[0, 0, 0] + out[1, 0, 0] - (4.0 * float(alpha) / 3.0)


def kernel(ent_emb, rel_emb, norm_vec, pos_triplets, neg_triplets):
  return _transh_loss(ent_emb, rel_emb, norm_vec, pos_triplets, neg_triplets,
                      margin=4.0, alpha=0.01)


# speedup vs baseline: 2.8351x; 2.2300x over previous
import functools

import jax
import jax.numpy as jnp
from jax.experimental import pallas as pl
from jax.experimental.pallas import tpu as pltpu

_SUB = 8


def _transh_kernel(
    ph_idx, pt_idx, nh_idx, nt_idx, pr_idx, nr_idx,
    ent_hbm,
    relnorm_ref,
    out_ref,
    ent_vmem,
    pht, ptt, nht, ntt,
    prt, nrt,
    copy_sem,
    *, margin, alpha, batch, dim, n_rows, cchunk):
  n_groups = n_rows // _SUB

  cp = pltpu.make_async_copy(ent_hbm, ent_vmem, copy_sem)
  cp.start()

  def rel_body(c, carry):
    base = c * _SUB
    for u in range(_SUB):
      gi = base + u
      prt[c, u] = relnorm_ref[pr_idx[gi], 0]
      nrt[c, u] = relnorm_ref[nr_idx[gi], 0]
    return carry
  jax.lax.fori_loop(0, n_groups, rel_body, 0)

  cp.wait()

  def ent_body(c, carry):
    base = c * _SUB
    for u in range(_SUB):
      gi = base + u
      pht[c, u] = ent_vmem[ph_idx[gi], 0]
      ptt[c, u] = ent_vmem[pt_idx[gi], 0]
      nht[c, u] = ent_vmem[nh_idx[gi], 0]
      ntt[c, u] = ent_vmem[nt_idx[gi], 0]
    return carry
  jax.lax.fori_loop(0, n_groups, ent_body, 0)

  cgroups = cchunk // _SUB
  n_cchunks = n_rows // cchunk

  def side(h, r, t, w):
    hw = jnp.sum(h * w, axis=2, keepdims=True)
    tw = jnp.sum(t * w, axis=2, keepdims=True)
    scores = (h + r - t) - (hw - tw) * w
    dist = jnp.sum(jnp.abs(scores), axis=2, keepdims=True)
    reg = jnp.sum(jnp.abs(h), axis=2, keepdims=True) + \
          jnp.sum(jnp.abs(t), axis=2, keepdims=True)
    r2 = jnp.sum(r * r, axis=2, keepdims=True)
    return dist, reg, r2

  def compute_body(c, carry):
    hinge_s, reg_s, r2_s = carry
    sl = pl.ds(c * cgroups, cgroups)
    ph, pt = pht[sl], ptt[sl]
    nh, nt = nht[sl], ntt[sl]
    prw, nrw = prt[sl], nrt[sl]
    pr, pw = prw[:, :, :dim], prw[:, :, dim:]
    nr, nw = nrw[:, :, :dim], nrw[:, :, dim:]

    pd, p_reg, p_r2 = side(ph, pr, pt, pw)
    nd, n_reg, n_r2 = side(nh, nr, nt, nw)

    rows = (c * cchunk
            + _SUB * jax.lax.broadcasted_iota(jnp.int32, (cgroups, _SUB, 1), 0)
            + jax.lax.broadcasted_iota(jnp.int32, (cgroups, _SUB, 1), 1))
    mask = (rows < batch).astype(jnp.float32)
    hinge = jnp.maximum(pd - nd + margin, 0.0)
    return (hinge_s + jnp.sum(hinge * mask),
            reg_s + jnp.sum((p_reg + n_reg) * mask),
            r2_s + jnp.sum((p_r2 + n_r2) * mask))

  zero = jnp.float32(0.0)
  hinge_s, reg_s, r2_s = jax.lax.fori_loop(
      0, n_cchunks, compute_body, (zero, zero, zero))

  inv_b = 1.0 / batch
  s = (hinge_s * inv_b
       + (alpha / 3.0) * (reg_s * inv_b + r2_s * (inv_b / dim)))
  out_ref[...] = jnp.reshape(s, (1, 1, 1))


def _transh_loss(ent_emb, rel_emb, norm_vec, pos_triplets, neg_triplets,
                 *, margin=4.0, alpha=0.01):
  B = int(pos_triplets.shape[0])
  E, D = int(ent_emb.shape[0]), int(ent_emb.shape[1])
  R = int(rel_emb.shape[0])

  cchunk = 256
  n_rows = pl.cdiv(B, cchunk) * cchunk
  n_groups = n_rows // _SUB

  ent3 = ent_emb.astype(jnp.float32).reshape(E, 1, D)
  relnorm = jnp.concatenate(
      [rel_emb.astype(jnp.float32), norm_vec.astype(jnp.float32)],
      axis=1).reshape(R, 1, 2 * D)

  def col(trip, j):
    c = trip[:, j].astype(jnp.int32)
    return jnp.pad(c, (0, n_rows - B))

  ph, pr, pt = col(pos_triplets, 0), col(pos_triplets, 1), col(pos_triplets, 2)
  nh, nr, nt = col(neg_triplets, 0), col(neg_triplets, 1), col(neg_triplets, 2)

  tiles_bytes = n_rows * (4 * D + 2 * 2 * D) * 4
  vmem_bytes = (E * D + R * 2 * D) * 4 + tiles_bytes + (8 << 20)
  grid_spec = pltpu.PrefetchScalarGridSpec(
      num_scalar_prefetch=6,
      grid=(1,),
      in_specs=[pl.BlockSpec(memory_space=pl.ANY),
                pl.BlockSpec((R, 1, 2 * D), lambda c, *_: (0, 0, 0))],
      out_specs=pl.BlockSpec((1, 1, 1), lambda c, *_: (0, 0, 0)),
      scratch_shapes=[
          pltpu.VMEM((E, 1, D), jnp.float32),
          pltpu.VMEM((n_groups, _SUB, D), jnp.float32),
          pltpu.VMEM((n_groups, _SUB, D), jnp.float32),
          pltpu.VMEM((n_groups, _SUB, D), jnp.float32),
          pltpu.VMEM((n_groups, _SUB, D), jnp.float32),
          pltpu.VMEM((n_groups, _SUB, 2 * D), jnp.float32),
          pltpu.VMEM((n_groups, _SUB, 2 * D), jnp.float32),
          pltpu.SemaphoreType.DMA,
      ])
  out = pl.pallas_call(
      functools.partial(_transh_kernel, margin=float(margin),
                        alpha=float(alpha), batch=B, dim=D,
                        n_rows=n_rows, cchunk=cchunk),
      out_shape=jax.ShapeDtypeStruct((1, 1, 1), jnp.float32),
      grid_spec=grid_spec,
      compiler_params=pltpu.CompilerParams(
          dimension_semantics=("arbitrary",),
          vmem_limit_bytes=int(min(58 * 2**20, vmem_bytes))),
      cost_estimate=pl.CostEstimate(
          flops=2 * n_rows * D * 30,
          transcendentals=0,
          bytes_accessed=(E * D + R * 2 * D + 4 * n_rows * D
                          + 6 * n_rows) * 4),
      name="transh_loss",
  )(ph, pt, nh, nt, pr, nr, ent3, relnorm)

  return out[0, 0, 0] - (4.0 * float(alpha) / 3.0)


def kernel(ent_emb, rel_emb, norm_vec, pos_triplets, neg_triplets):
  return _transh_loss(ent_emb, rel_emb, norm_vec, pos_triplets, neg_triplets,
                      margin=4.0, alpha=0.01)


# speedup vs baseline: 3.1082x; 1.0963x over previous
import functools

import jax
import jax.numpy as jnp
from jax.experimental import pallas as pl
from jax.experimental.pallas import tpu as pltpu

_SUB = 8


def _transh_kernel(
    ph_idx, pt_idx, nh_idx, nt_idx, pr_idx, nr_idx,
    ent_hbm,
    relnorm_ref,
    out_ref,
    ent_vmem,
    pht, ptt, nht, ntt,
    prt, nrt,
    copy_sem,
    *, margin, alpha, batch, dim, n_rows, cchunk):
  n_groups = n_rows // _SUB

  cp = pltpu.make_async_copy(ent_hbm, ent_vmem, copy_sem)
  cp.start()

  def rel_body(c, carry):
    base = c * _SUB
    for u in range(_SUB):
      gi = base + u
      prt[c, u] = relnorm_ref[pr_idx[gi], 0]
      nrt[c, u] = relnorm_ref[nr_idx[gi], 0]
    return carry
  jax.lax.fori_loop(0, n_groups, rel_body, 0)

  cp.wait()

  def ent_body(c, carry):
    base = c * _SUB
    for u in range(_SUB):
      gi = base + u
      pht[c, u] = ent_vmem[ph_idx[gi], 0]
      ptt[c, u] = ent_vmem[pt_idx[gi], 0]
      nht[c, u] = ent_vmem[nh_idx[gi], 0]
      ntt[c, u] = ent_vmem[nt_idx[gi], 0]
    return carry
  jax.lax.fori_loop(0, n_groups, ent_body, 0)

  cgroups = cchunk // _SUB
  n_cchunks = n_rows // cchunk

  inv_dim = 1.0 / dim

  def side(h, r, t, w):
    d = h - t
    dw = jnp.sum(d * w, axis=2, keepdims=True)
    scores = (d + r) - dw * w
    dist = jnp.sum(jnp.abs(scores), axis=2, keepdims=True)
    q = jnp.sum(jnp.abs(h) + jnp.abs(t) + (r * r) * inv_dim,
                axis=2, keepdims=True)
    return dist, q

  def compute_body(c, carry):
    hinge_s, q_s = carry
    sl = pl.ds(c * cgroups, cgroups)
    ph, pt = pht[sl], ptt[sl]
    nh, nt = nht[sl], ntt[sl]
    prw, nrw = prt[sl], nrt[sl]
    pr, pw = prw[:, :, :dim], prw[:, :, dim:]
    nr, nw = nrw[:, :, :dim], nrw[:, :, dim:]

    pd, p_q = side(ph, pr, pt, pw)
    nd, n_q = side(nh, nr, nt, nw)

    rows = (c * cchunk
            + _SUB * jax.lax.broadcasted_iota(jnp.int32, (cgroups, _SUB, 1), 0)
            + jax.lax.broadcasted_iota(jnp.int32, (cgroups, _SUB, 1), 1))
    mask = (rows < batch).astype(jnp.float32)
    hinge = jnp.maximum(pd - nd + margin, 0.0)
    return (hinge_s + jnp.sum(hinge * mask),
            q_s + jnp.sum((p_q + n_q) * mask))

  zero = jnp.float32(0.0)
  hinge_s, q_s = jax.lax.fori_loop(
      0, n_cchunks, compute_body, (zero, zero))

  inv_b = 1.0 / batch
  s = hinge_s * inv_b + (alpha / 3.0) * (q_s * inv_b)
  out_ref[...] = jnp.reshape(s, (1, 1, 1))


def _transh_loss(ent_emb, rel_emb, norm_vec, pos_triplets, neg_triplets,
                 *, margin=4.0, alpha=0.01):
  B = int(pos_triplets.shape[0])
  E, D = int(ent_emb.shape[0]), int(ent_emb.shape[1])
  R = int(rel_emb.shape[0])

  cchunk = 256
  n_rows = pl.cdiv(B, cchunk) * cchunk
  n_groups = n_rows // _SUB

  ent3 = ent_emb.astype(jnp.float32).reshape(E, 1, D)
  relnorm = jnp.concatenate(
      [rel_emb.astype(jnp.float32), norm_vec.astype(jnp.float32)],
      axis=1).reshape(R, 1, 2 * D)

  def col(trip, j):
    c = trip[:, j].astype(jnp.int32)
    return jnp.pad(c, (0, n_rows - B))

  ph, pr, pt = col(pos_triplets, 0), col(pos_triplets, 1), col(pos_triplets, 2)
  nh, nr, nt = col(neg_triplets, 0), col(neg_triplets, 1), col(neg_triplets, 2)

  tiles_bytes = n_rows * (4 * D + 2 * 2 * D) * 4
  vmem_bytes = (E * D + R * 2 * D) * 4 + tiles_bytes + (8 << 20)
  grid_spec = pltpu.PrefetchScalarGridSpec(
      num_scalar_prefetch=6,
      grid=(1,),
      in_specs=[pl.BlockSpec(memory_space=pl.ANY),
                pl.BlockSpec((R, 1, 2 * D), lambda c, *_: (0, 0, 0))],
      out_specs=pl.BlockSpec((1, 1, 1), lambda c, *_: (0, 0, 0)),
      scratch_shapes=[
          pltpu.VMEM((E, 1, D), jnp.float32),
          pltpu.VMEM((n_groups, _SUB, D), jnp.float32),
          pltpu.VMEM((n_groups, _SUB, D), jnp.float32),
          pltpu.VMEM((n_groups, _SUB, D), jnp.float32),
          pltpu.VMEM((n_groups, _SUB, D), jnp.float32),
          pltpu.VMEM((n_groups, _SUB, 2 * D), jnp.float32),
          pltpu.VMEM((n_groups, _SUB, 2 * D), jnp.float32),
          pltpu.SemaphoreType.DMA,
      ])
  out = pl.pallas_call(
      functools.partial(_transh_kernel, margin=float(margin),
                        alpha=float(alpha), batch=B, dim=D,
                        n_rows=n_rows, cchunk=cchunk),
      out_shape=jax.ShapeDtypeStruct((1, 1, 1), jnp.float32),
      grid_spec=grid_spec,
      compiler_params=pltpu.CompilerParams(
          dimension_semantics=("arbitrary",),
          vmem_limit_bytes=int(min(58 * 2**20, vmem_bytes))),
      cost_estimate=pl.CostEstimate(
          flops=2 * n_rows * D * 30,
          transcendentals=0,
          bytes_accessed=(E * D + R * 2 * D + 4 * n_rows * D
                          + 6 * n_rows) * 4),
      name="transh_loss",
  )(ph, pt, nh, nt, pr, nr, ent3, relnorm)

  return out[0, 0, 0] - (4.0 * float(alpha) / 3.0)


def kernel(ent_emb, rel_emb, norm_vec, pos_triplets, neg_triplets):
  return _transh_loss(ent_emb, rel_emb, norm_vec, pos_triplets, neg_triplets,
                      margin=4.0, alpha=0.01)
